# baseline (speedup 1.0000x reference)
import functools
import jax
import jax.numpy as jnp
from jax import lax
from jax.experimental import pallas as pl
from jax.experimental.pallas import tpu as pltpu
from jax.experimental.pallas import tpu_sc as plsc

D_DIM = 1024
CHUNK_ROWS = 16
CHUNK_W = CHUNK_ROWS * D_DIM
N_WORKERS = 32
UNROLL = 8


def _sc_body(x_hbm, e_hbm, o_hbm, e_v, x_v):
    wid = lax.axis_index("s") * 2 + lax.axis_index("c")
    rows_per_w = 4096 // N_WORKERS
    nchunks = rows_per_w // CHUNK_ROWS

    def chunk_body(k, carry):
        s0 = wid * rows_per_w + k * CHUNK_ROWS
        pltpu.sync_copy(e_hbm.at[pl.ds(s0 * D_DIM, CHUNK_W)], e_v)
        for b in range(4):
            xoff = (b * 4096 + s0) * D_DIM
            pltpu.sync_copy(x_hbm.at[pl.ds(xoff, CHUNK_W)], x_v)

            def add_body(j, c2):
                for u in range(UNROLL):
                    sl = pl.ds((j * UNROLL + u) * 16, 16)
                    x_v[sl] = x_v[sl] + e_v[sl]
                return c2

            lax.fori_loop(0, CHUNK_W // 16 // UNROLL, add_body, 0)
            pltpu.sync_copy(x_v, o_hbm.at[pl.ds(xoff, CHUNK_W)])
        return carry

    lax.fori_loop(0, nchunks, chunk_body, 0)


def kernel(x, embed_weight):
    B, S, D = x.shape
    xf = x.reshape(-1)
    ef = embed_weight.reshape(-1)
    mesh = plsc.VectorSubcoreMesh(core_axis_name="c", subcore_axis_name="s")
    f = pl.kernel(
        _sc_body,
        out_type=jax.ShapeDtypeStruct((B * S * D,), jnp.float32),
        mesh=mesh,
        scratch_types=[
            pltpu.VMEM((CHUNK_W,), jnp.float32),
            pltpu.VMEM((CHUNK_W,), jnp.float32),
        ],
    )
    return f(xf, ef).reshape(B, S, D)


# speedup vs baseline: 1.2443x; 1.2443x over previous
import jax
import jax.numpy as jnp
from jax import lax
from jax.experimental import pallas as pl
from jax.experimental.pallas import tpu as pltpu
from jax.experimental.pallas import tpu_sc as plsc

D_DIM = 1024
CHUNK_ROWS = 16
CHUNK_W = CHUNK_ROWS * D_DIM
N_WORKERS = 32
UNROLL = 8
NBUF = 4


def _sc_body(x_hbm, e_hbm, o_hbm, *refs):
    ev = list(refs[0:2])
    xv = list(refs[2:6])
    esem = list(refs[6:8])
    xisem = list(refs[8:12])
    xosem = list(refs[12:16])

    wid = lax.axis_index("s") * 2 + lax.axis_index("c")
    rows = 4096 // N_WORKERS
    nch = rows // CHUNK_ROWS
    nsteps = nch * 4
    base_s = wid * rows

    def e_desc(k):
        s0 = base_s + k * CHUNK_ROWS
        return pltpu.make_async_copy(
            e_hbm.at[pl.ds(s0 * D_DIM, CHUNK_W)], ev[k % 2], esem[k % 2])

    def x_off(t):
        k, b = divmod(t, 4)
        s0 = base_s + k * CHUNK_ROWS
        return (b * 4096 + s0) * D_DIM

    def xin_desc(t):
        return pltpu.make_async_copy(
            x_hbm.at[pl.ds(x_off(t), CHUNK_W)], xv[t % NBUF], xisem[t % NBUF])

    def xout_desc(t):
        return pltpu.make_async_copy(
            xv[t % NBUF], o_hbm.at[pl.ds(x_off(t), CHUNK_W)], xosem[t % NBUF])

    e_desc(0).start()
    e_desc(1).start()
    xin_desc(0).start()
    xin_desc(1).start()

    for t in range(nsteps):
        k, b = divmod(t, 4)
        p = t % NBUF
        if b == 0:
            e_desc(k).wait()
        xin_desc(t).wait()
        xb, eb = xv[p], ev[k % 2]

        def add_body(j, c2, xb=xb, eb=eb):
            for u in range(UNROLL):
                sl = pl.ds((j * UNROLL + u) * 16, 16)
                xb[sl] = xb[sl] + eb[sl]
            return c2

        lax.fori_loop(0, CHUNK_W // 16 // UNROLL, add_body, 0)
        xout_desc(t).start()
        if b == 3 and k + 2 < nch:
            e_desc(k + 2).start()
        if t + 2 < nsteps:
            if t - 2 >= 0:
                xout_desc(t - 2).wait()
            xin_desc(t + 2).start()

    xout_desc(nsteps - 2).wait()
    xout_desc(nsteps - 1).wait()


def kernel(x, embed_weight):
    B, S, D = x.shape
    xf = x.reshape(-1)
    ef = embed_weight.reshape(-1)
    mesh = plsc.VectorSubcoreMesh(core_axis_name="c", subcore_axis_name="s")
    scratch = (
        [pltpu.VMEM((CHUNK_W,), jnp.float32)] * 2
        + [pltpu.VMEM((CHUNK_W,), jnp.float32)] * 4
        + [pltpu.SemaphoreType.DMA] * 10
    )
    f = pl.kernel(
        _sc_body,
        out_type=jax.ShapeDtypeStruct((B * S * D,), jnp.float32),
        mesh=mesh,
        scratch_types=scratch,
    )
    return f(xf, ef).reshape(B, S, D)
